# Initial kernel scaffold; baseline (speedup 1.0000x reference)
import jax
import jax.numpy as jnp
from jax.experimental import pallas as pl


def kernel(batch, src_weight, dst_weight):
    raise NotImplementedError("write your pallas kernel here")


# speedup vs baseline: 1.4303x; 1.4303x over previous
import jax
import jax.numpy as jnp
from jax.experimental import pallas as pl
from jax.experimental.pallas import tpu as pltpu
from jax.experimental.pallas import tpu_sc as plsc

_EMBED_DIM = 128
_WINDOW = 128


def kernel(batch, src_weight, dst_weight):
    batch = batch.astype(jnp.int32)
    n = batch.shape[0]
    indices = batch.reshape(1, n)
    out_sd = jax.ShapeDtypeStruct((n, _EMBED_DIM), src_weight.dtype)

    mesh = plsc.VectorSubcoreMesh(core_axis_name="core", subcore_axis_name="subcore")

    @jax.jit
    @pl.kernel(out_type=(out_sd, out_sd), mesh=mesh)
    def gather2(src_hbm, dst_hbm, i_hbm, o_src_hbm, o_dst_hbm):
        def body(i_vmem, o_src_vmem, o_dst_vmem):
            pltpu.sync_copy(src_hbm.at[i_vmem.at[0]], o_src_vmem)
            pltpu.sync_copy(dst_hbm.at[i_vmem.at[0]], o_dst_vmem)

        pltpu.emit_pipeline(
            body,
            grid=(n // _WINDOW,),
            in_specs=[pl.BlockSpec((1, _WINDOW), index_map=lambda i: (0, i))],
            out_specs=[
                pl.BlockSpec((_WINDOW, _EMBED_DIM), index_map=lambda i: (i, 0)),
                pl.BlockSpec((_WINDOW, _EMBED_DIM), index_map=lambda i: (i, 0)),
            ],
            core_axis_name=("core", "subcore"),
            dimension_semantics=(pltpu.PARALLEL,),
        )(i_hbm, o_src_hbm, o_dst_hbm)

    return gather2(src_weight, dst_weight, indices)


# speedup vs baseline: 1.5135x; 1.0581x over previous
import jax
import jax.numpy as jnp
from jax.experimental import pallas as pl
from jax.experimental.pallas import tpu as pltpu
from jax.experimental.pallas import tpu_sc as plsc

_EMBED_DIM = 128
_WINDOW = 128


def kernel(batch, src_weight, dst_weight):
    batch = batch.astype(jnp.int32)
    n = batch.shape[0]
    indices = batch.reshape(1, n)
    out_sd = jax.ShapeDtypeStruct((n, _EMBED_DIM), src_weight.dtype)

    mesh = plsc.VectorSubcoreMesh(core_axis_name="core", subcore_axis_name="subcore")

    @jax.jit
    @pl.kernel(
        out_type=(out_sd, out_sd),
        mesh=mesh,
        scratch_types=[pltpu.SemaphoreType.DMA, pltpu.SemaphoreType.DMA],
    )
    def gather2(src_hbm, dst_hbm, i_hbm, o_src_hbm, o_dst_hbm, sem_a, sem_b):
        def body(i_vmem, o_src_vmem, o_dst_vmem):
            cp_a = pltpu.async_copy(src_hbm.at[i_vmem.at[0]], o_src_vmem, sem_a)
            cp_b = pltpu.async_copy(dst_hbm.at[i_vmem.at[0]], o_dst_vmem, sem_b)
            cp_a.wait()
            cp_b.wait()

        pltpu.emit_pipeline(
            body,
            grid=(n // _WINDOW,),
            in_specs=[pl.BlockSpec((1, _WINDOW), index_map=lambda i: (0, i))],
            out_specs=[
                pl.BlockSpec((_WINDOW, _EMBED_DIM), index_map=lambda i: (i, 0)),
                pl.BlockSpec((_WINDOW, _EMBED_DIM), index_map=lambda i: (i, 0)),
            ],
            core_axis_name=("core", "subcore"),
            dimension_semantics=(pltpu.PARALLEL,),
        )(i_hbm, o_src_hbm, o_dst_hbm)

    return gather2(src_weight, dst_weight, indices)
